# baseline (speedup 1.0000x reference)
import functools

import jax
import jax.numpy as jnp
from jax import lax
from jax.experimental import pallas as pl
from jax.experimental.pallas import tpu as pltpu
from jax.experimental.pallas import tpu_sc as plsc

N = 10000
HID = 16
OUT_DIM = 64
EPS = 1e-5

NC, NS, LANES = 2, 16, 16
NW = NC * NS
CHUNK = 128
RPS = 632
NPAD = NS * RPS


GB = 2048


def _sc_degree(dst_idx):
    KX = dst_idx.shape[1]
    K = KX - GB // CHUNK
    mesh = plsc.VectorSubcoreMesh(core_axis_name="c", subcore_axis_name="s",
                                  num_cores=NC, num_subcores=NS)

    @functools.partial(
        pl.kernel, mesh=mesh,
        out_type=jax.ShapeDtypeStruct((NC, NPAD, LANES), jnp.float32),
        scratch_types=[
            pltpu.VMEM((KX, CHUNK), jnp.int32),
            pltpu.VMEM((CHUNK, LANES), jnp.float32),
            pltpu.VMEM((RPS, LANES), jnp.float32),
            pltpu.VMEM_SHARED((NPAD, LANES), jnp.float32),
            pltpu.SemaphoreType.DMA,
        ],
        compiler_params=pltpu.CompilerParams(use_tc_tiling_on_sc=False))
    def k(dst_hbm, out_hbm, dstv, onesb, zbuf, acc, sem):
        c = lax.axis_index("c")
        s = lax.axis_index("s")
        wid = c * NS + s

        def fill_zero(i, carry):
            zbuf[i, :] = jnp.zeros((LANES,), jnp.float32)
            return carry
        lax.fori_loop(0, RPS, fill_zero, None)

        def fill_one(i, carry):
            onesb[i, :] = jnp.ones((LANES,), jnp.float32)
            return carry
        lax.fori_loop(0, CHUNK, fill_one, None)

        pltpu.sync_copy(zbuf, acc.at[pl.ds(s * RPS, RPS)])
        pltpu.sync_copy(dst_hbm.at[wid], dstv)
        plsc.subcore_barrier()

        def body(j, carry):
            pltpu.sync_copy(onesb, acc.at[dstv.at[j]], add=True)
            return carry
        lax.fori_loop(0, K, body, None)

        plsc.subcore_barrier()
        pltpu.sync_copy(acc.at[pl.ds(s * RPS, RPS)],
                        out_hbm.at[c, pl.ds(s * RPS, RPS)])

    return k(dst_idx)


def _sc_scatter_rows(src_idx, dst_idx, table):
    KX = dst_idx.shape[1]
    SPG = GB // CHUNK
    G = KX * CHUNK // GB - 1
    mesh = plsc.VectorSubcoreMesh(core_axis_name="c", subcore_axis_name="s",
                                  num_cores=NC, num_subcores=NS)

    @functools.partial(
        pl.kernel, mesh=mesh,
        out_type=jax.ShapeDtypeStruct((NC, NPAD, LANES), jnp.float32),
        scratch_types=[
            pltpu.VMEM((KX * CHUNK,), jnp.int32),
            pltpu.VMEM((KX, CHUNK), jnp.int32),
            pltpu.VMEM((2, GB, LANES), jnp.float32),
            pltpu.VMEM((RPS, LANES), jnp.float32),
            pltpu.VMEM_SHARED((NPAD, LANES), jnp.float32),
            pltpu.SemaphoreType.DMA,
            pltpu.SemaphoreType.DMA,
        ],
        compiler_params=pltpu.CompilerParams(use_tc_tiling_on_sc=False))
    def k(src_hbm, dst_hbm, tab_hbm, out_hbm, srcf, dstv, rows, zbuf,
          acc, gsemA, gsemB):
        c = lax.axis_index("c")
        s = lax.axis_index("s")
        wid = c * NS + s

        def fill_zero(i, carry):
            zbuf[i, :] = jnp.zeros((LANES,), jnp.float32)
            return carry
        lax.fori_loop(0, RPS, fill_zero, None)

        pltpu.sync_copy(zbuf, acc.at[pl.ds(s * RPS, RPS)])
        pltpu.sync_copy(src_hbm.at[wid], srcf)
        pltpu.sync_copy(dst_hbm.at[wid], dstv)
        plsc.subcore_barrier()

        sems = (gsemA, gsemB)

        def gather(g, b):
            return pltpu.async_copy(
                tab_hbm.at[srcf.at[pl.ds(g * GB, GB)]], rows.at[b], sems[b])

        def gather_wait(g, b):
            pltpu.make_async_copy(tab_hbm.at[srcf.at[pl.ds(g * GB, GB)]],
                                  rows.at[b], sems[b]).wait()

        def scatters(g, b):
            for t in range(SPG):
                pltpu.sync_copy(rows.at[b, pl.ds(t * CHUNK, CHUNK)],
                                acc.at[dstv.at[g * SPG + t]], add=True)

        gather(0, 0)
        gather(1, 1)

        def body(h, carry):
            g = 2 * h
            gather_wait(g, 0)
            gather(g + 2, 0)
            scatters(g, 0)
            gather_wait(g + 1, 1)
            gather(g + 3, 1)
            scatters(g + 1, 1)
            return carry
        lax.fori_loop(0, (G - 1) // 2, body, None)

        gather_wait(G - 1, 0)
        scatters(G - 1, 0)
        gather_wait(G, 1)

        plsc.subcore_barrier()
        pltpu.sync_copy(acc.at[pl.ds(s * RPS, RPS)],
                        out_hbm.at[c, pl.ds(s * RPS, RPS)])

    return k(src_idx, dst_idx, table)


def _tc_prep(xp, W1, dacc):
    def body(x_ref, w_ref, da_ref, db_ref, dinv_ref, dhv_ref):
        deg = da_ref[...] + db_ref[...] + 1.0
        rows = lax.broadcasted_iota(jnp.int32, (NPAD, HID), 0)
        dinv = jnp.where(rows < N, lax.rsqrt(deg), 0.0)
        h = jnp.dot(x_ref[...], w_ref[...], preferred_element_type=jnp.float32)
        dinv_ref[...] = dinv
        dhv_ref[...] = dinv * h

    return pl.pallas_call(
        body,
        out_shape=(jax.ShapeDtypeStruct((NPAD, HID), jnp.float32),
                   jax.ShapeDtypeStruct((NPAD, HID), jnp.float32)),
    )(xp, W1, dacc[0], dacc[1])


def _tc_mid(acc, dhv, dinv, b, g, be, W):
    def body(a_ref, b2_ref, dhv_ref, dinv_ref, bias_ref, g_ref, be_ref, w_ref,
             out_ref):
        rows = lax.broadcasted_iota(jnp.int32, (NPAD, HID), 0)
        valid = rows < N
        dinv = dinv_ref[...]
        s = dinv * (a_ref[...] + b2_ref[...] + dhv_ref[...]) + bias_ref[...]
        sv = jnp.where(valid, s, 0.0)
        mean = jnp.sum(sv, axis=0, keepdims=True) * (1.0 / N)
        d = s - mean
        var = jnp.sum(jnp.where(valid, d * d, 0.0), axis=0, keepdims=True) * (1.0 / N)
        bn = d * lax.rsqrt(var + EPS) * g_ref[...] + be_ref[...]
        h = jnp.where(valid, jnp.maximum(bn, 0.0), 0.0)
        out_ref[...] = dinv * jnp.dot(h, w_ref[...],
                                      preferred_element_type=jnp.float32)

    return pl.pallas_call(
        body,
        out_shape=jax.ShapeDtypeStruct((NPAD, HID), jnp.float32),
    )(acc[0], acc[1], dhv, dinv, b, g, be, W)


def _tc_final(acc, dhv, dinv, b, g, be, Wfc, bfc):
    def body(a_ref, b2_ref, dhv_ref, dinv_ref, bias_ref, g_ref, be_ref, w_ref,
             bf_ref, out_ref):
        rows = lax.broadcasted_iota(jnp.int32, (NPAD, HID), 0)
        valid = rows < N
        s = dinv_ref[...] * (a_ref[...] + b2_ref[...] + dhv_ref[...]) + bias_ref[...]
        sv = jnp.where(valid, s, 0.0)
        mean = jnp.sum(sv, axis=0, keepdims=True) * (1.0 / N)
        d = s - mean
        var = jnp.sum(jnp.where(valid, d * d, 0.0), axis=0, keepdims=True) * (1.0 / N)
        bn = d * lax.rsqrt(var + EPS) * g_ref[...] + be_ref[...]
        h = jnp.where(valid, jnp.maximum(bn, 0.0), 0.0)
        out_ref[...] = jnp.dot(h, w_ref[...],
                               preferred_element_type=jnp.float32) + bf_ref[...]

    return pl.pallas_call(
        body,
        out_shape=jax.ShapeDtypeStruct((NPAD, OUT_DIM), jnp.float32),
    )(acc[0], acc[1], dhv, dinv, b, g, be, Wfc, bfc)


def kernel(x, edge_index, W1, b1, g1, be1, W2, b2, g2, be2, Wfc, bfc):
    E = edge_index.shape[1]
    SPG = GB // CHUNK
    K = -(-E // (NW * CHUNK))
    K = -(-K // SPG) * SPG
    if (K // SPG) % 2 == 0:
        K += SPG
    KX = K + SPG
    EP = NW * CHUNK * K

    ei = edge_index.astype(jnp.int32)
    pad = jnp.full((EP - E,), NPAD - 1, jnp.int32)
    runout = jnp.full((NW, SPG, CHUNK), NPAD - 1, jnp.int32)
    src = jnp.concatenate(
        [jnp.concatenate([ei[0], pad]).reshape(NW, K, CHUNK), runout], axis=1)
    dst = jnp.concatenate(
        [jnp.concatenate([ei[1], pad]).reshape(NW, K, CHUNK), runout], axis=1)
    xp = jnp.pad(x, ((0, NPAD - N), (0, 0)))

    srcf = src.reshape(NW, KX * CHUNK)
    dacc = _sc_degree(dst)
    dinv, ht1 = _tc_prep(xp, W1, dacc)
    acc1 = _sc_scatter_rows(srcf, dst, ht1)
    ht2 = _tc_mid(acc1, ht1, dinv, b1.reshape(1, -1), g1.reshape(1, -1),
                  be1.reshape(1, -1), W2)
    acc2 = _sc_scatter_rows(srcf, dst, ht2)
    out = _tc_final(acc2, ht2, dinv, b2.reshape(1, -1), g2.reshape(1, -1),
                    be2.reshape(1, -1), Wfc, bfc.reshape(1, -1))
    return out[:N]


# speedup vs baseline: 4.0332x; 4.0332x over previous
import functools

import jax
import jax.numpy as jnp
from jax import lax
from jax.experimental import pallas as pl
from jax.experimental.pallas import tpu as pltpu
from jax.experimental.pallas import tpu_sc as plsc

N = 10000
HID = 16
OUT_DIM = 64
EPS = 1e-5

NC, NS, LANES = 2, 16, 16
NW = NC * NS
CHUNK = 128
RPS = 632
NPAD = NS * RPS


GB = 2048


def _sc_degree(dst_idx):
    KX = dst_idx.shape[1]
    K = KX - GB // CHUNK
    mesh = plsc.VectorSubcoreMesh(core_axis_name="c", subcore_axis_name="s",
                                  num_cores=NC, num_subcores=NS)

    @functools.partial(
        pl.kernel, mesh=mesh,
        out_type=jax.ShapeDtypeStruct((NC, NPAD, LANES), jnp.float32),
        scratch_types=[
            pltpu.VMEM((KX, CHUNK), jnp.int32),
            pltpu.VMEM((CHUNK, LANES), jnp.float32),
            pltpu.VMEM((RPS, LANES), jnp.float32),
            pltpu.VMEM_SHARED((NPAD, LANES), jnp.float32),
            pltpu.SemaphoreType.DMA,
        ],
        compiler_params=pltpu.CompilerParams(use_tc_tiling_on_sc=False))
    def k(dst_hbm, out_hbm, dstv, onesb, zbuf, acc, sem):
        c = lax.axis_index("c")
        s = lax.axis_index("s")
        wid = c * NS + s

        def fill_zero(i, carry):
            zbuf[i, :] = jnp.zeros((LANES,), jnp.float32)
            return carry
        lax.fori_loop(0, RPS, fill_zero, None)

        def fill_one(i, carry):
            onesb[i, :] = jnp.ones((LANES,), jnp.float32)
            return carry
        lax.fori_loop(0, CHUNK, fill_one, None)

        pltpu.sync_copy(zbuf, acc.at[pl.ds(s * RPS, RPS)])
        pltpu.sync_copy(dst_hbm.at[wid], dstv)
        plsc.subcore_barrier()

        def body(j, carry):
            pltpu.sync_copy(onesb, acc.at[dstv.at[j]], add=True)
            return carry
        lax.fori_loop(0, K, body, None)

        plsc.subcore_barrier()
        pltpu.sync_copy(acc.at[pl.ds(s * RPS, RPS)],
                        out_hbm.at[c, pl.ds(s * RPS, RPS)])

    return k(dst_idx)


def _sc_scatter_rows(src_idx, dst_idx, table):
    KX = dst_idx.shape[1]
    K = KX - GB // CHUNK
    mesh = plsc.VectorSubcoreMesh(core_axis_name="c", subcore_axis_name="s",
                                  num_cores=NC, num_subcores=NS)

    @functools.partial(
        pl.kernel, mesh=mesh,
        out_type=jax.ShapeDtypeStruct((NC, NPAD, LANES), jnp.float32),
        scratch_types=[
            pltpu.VMEM((KX, CHUNK), jnp.int32),
            pltpu.VMEM((KX, CHUNK), jnp.int32),
            pltpu.VMEM((CHUNK, LANES), jnp.float32),
            pltpu.VMEM((RPS, LANES), jnp.float32),
            pltpu.VMEM_SHARED((NPAD, LANES), jnp.float32),
            pltpu.VMEM_SHARED((NPAD, LANES), jnp.float32),
            pltpu.SemaphoreType.DMA,
        ],
        compiler_params=pltpu.CompilerParams(use_tc_tiling_on_sc=False))
    def k(src_hbm, dst_hbm, tab_hbm, out_hbm, srcv, dstv, rows, zbuf,
          acc, tabs, gsem):
        c = lax.axis_index("c")
        s = lax.axis_index("s")
        wid = c * NS + s

        def fill_zero(i, carry):
            zbuf[i, :] = jnp.zeros((LANES,), jnp.float32)
            return carry
        lax.fori_loop(0, RPS, fill_zero, None)

        pltpu.sync_copy(zbuf, acc.at[pl.ds(s * RPS, RPS)])
        pltpu.sync_copy(tab_hbm.at[pl.ds(s * RPS, RPS)],
                        tabs.at[pl.ds(s * RPS, RPS)])
        pltpu.sync_copy(src_hbm.at[wid], srcv)
        pltpu.sync_copy(dst_hbm.at[wid], dstv)
        plsc.subcore_barrier()

        def body(j, carry):
            pltpu.async_copy(tabs.at[srcv.at[j]], rows, gsem).wait()
            pltpu.sync_copy(rows, acc.at[dstv.at[j]], add=True)
            return carry
        lax.fori_loop(0, K, body, None)

        plsc.subcore_barrier()
        pltpu.sync_copy(acc.at[pl.ds(s * RPS, RPS)],
                        out_hbm.at[c, pl.ds(s * RPS, RPS)])

    return k(src_idx, dst_idx, table)


def _tc_prep(xp, W1, dacc):
    def body(x_ref, w_ref, da_ref, db_ref, dinv_ref, dhv_ref):
        deg = da_ref[...] + db_ref[...] + 1.0
        rows = lax.broadcasted_iota(jnp.int32, (NPAD, HID), 0)
        dinv = jnp.where(rows < N, lax.rsqrt(deg), 0.0)
        h = jnp.dot(x_ref[...], w_ref[...], preferred_element_type=jnp.float32)
        dinv_ref[...] = dinv
        dhv_ref[...] = dinv * h

    return pl.pallas_call(
        body,
        out_shape=(jax.ShapeDtypeStruct((NPAD, HID), jnp.float32),
                   jax.ShapeDtypeStruct((NPAD, HID), jnp.float32)),
    )(xp, W1, dacc[0], dacc[1])


def _tc_mid(acc, dhv, dinv, b, g, be, W):
    def body(a_ref, b2_ref, dhv_ref, dinv_ref, bias_ref, g_ref, be_ref, w_ref,
             out_ref):
        rows = lax.broadcasted_iota(jnp.int32, (NPAD, HID), 0)
        valid = rows < N
        dinv = dinv_ref[...]
        s = dinv * (a_ref[...] + b2_ref[...] + dhv_ref[...]) + bias_ref[...]
        sv = jnp.where(valid, s, 0.0)
        mean = jnp.sum(sv, axis=0, keepdims=True) * (1.0 / N)
        d = s - mean
        var = jnp.sum(jnp.where(valid, d * d, 0.0), axis=0, keepdims=True) * (1.0 / N)
        bn = d * lax.rsqrt(var + EPS) * g_ref[...] + be_ref[...]
        h = jnp.where(valid, jnp.maximum(bn, 0.0), 0.0)
        out_ref[...] = dinv * jnp.dot(h, w_ref[...],
                                      preferred_element_type=jnp.float32)

    return pl.pallas_call(
        body,
        out_shape=jax.ShapeDtypeStruct((NPAD, HID), jnp.float32),
    )(acc[0], acc[1], dhv, dinv, b, g, be, W)


def _tc_final(acc, dhv, dinv, b, g, be, Wfc, bfc):
    def body(a_ref, b2_ref, dhv_ref, dinv_ref, bias_ref, g_ref, be_ref, w_ref,
             bf_ref, out_ref):
        rows = lax.broadcasted_iota(jnp.int32, (NPAD, HID), 0)
        valid = rows < N
        s = dinv_ref[...] * (a_ref[...] + b2_ref[...] + dhv_ref[...]) + bias_ref[...]
        sv = jnp.where(valid, s, 0.0)
        mean = jnp.sum(sv, axis=0, keepdims=True) * (1.0 / N)
        d = s - mean
        var = jnp.sum(jnp.where(valid, d * d, 0.0), axis=0, keepdims=True) * (1.0 / N)
        bn = d * lax.rsqrt(var + EPS) * g_ref[...] + be_ref[...]
        h = jnp.where(valid, jnp.maximum(bn, 0.0), 0.0)
        out_ref[...] = jnp.dot(h, w_ref[...],
                               preferred_element_type=jnp.float32) + bf_ref[...]

    return pl.pallas_call(
        body,
        out_shape=jax.ShapeDtypeStruct((NPAD, OUT_DIM), jnp.float32),
    )(acc[0], acc[1], dhv, dinv, b, g, be, Wfc, bfc)


def kernel(x, edge_index, W1, b1, g1, be1, W2, b2, g2, be2, Wfc, bfc):
    E = edge_index.shape[1]
    SPG = GB // CHUNK
    K = -(-E // (NW * CHUNK))
    K = -(-K // SPG) * SPG
    if (K // SPG) % 2 == 0:
        K += SPG
    KX = K + SPG
    EP = NW * CHUNK * K

    ei = edge_index.astype(jnp.int32)
    pad = jnp.full((EP - E,), NPAD - 1, jnp.int32)
    runout = jnp.full((NW, SPG, CHUNK), NPAD - 1, jnp.int32)
    src = jnp.concatenate(
        [jnp.concatenate([ei[0], pad]).reshape(NW, K, CHUNK), runout], axis=1)
    dst = jnp.concatenate(
        [jnp.concatenate([ei[1], pad]).reshape(NW, K, CHUNK), runout], axis=1)
    xp = jnp.pad(x, ((0, NPAD - N), (0, 0)))

    dacc = _sc_degree(dst)
    dinv, ht1 = _tc_prep(xp, W1, dacc)
    acc1 = _sc_scatter_rows(src, dst, ht1)
    ht2 = _tc_mid(acc1, ht1, dinv, b1.reshape(1, -1), g1.reshape(1, -1),
                  be1.reshape(1, -1), W2)
    acc2 = _sc_scatter_rows(src, dst, ht2)
    out = _tc_final(acc2, ht2, dinv, b2.reshape(1, -1), g2.reshape(1, -1),
                    be2.reshape(1, -1), Wfc, bfc.reshape(1, -1))
    return out[:N]


# speedup vs baseline: 4.1332x; 1.0248x over previous
import functools

import jax
import jax.numpy as jnp
from jax import lax
from jax.experimental import pallas as pl
from jax.experimental.pallas import tpu as pltpu
from jax.experimental.pallas import tpu_sc as plsc

N = 10000
HID = 16
OUT_DIM = 64
EPS = 1e-5

NC, NS, LANES = 2, 16, 16
NW = NC * NS
CHUNK = 128
RPS = 632
NPAD = NS * RPS


def _worker_range(wid, cn):
    start = wid * cn // NW
    end = (wid + 1) * cn // NW
    return start, end


def _sc_degree(dst_idx):
    CN = dst_idx.shape[0]
    MAXCH = -(-CN // NW)
    mesh = plsc.VectorSubcoreMesh(core_axis_name="c", subcore_axis_name="s",
                                  num_cores=NC, num_subcores=NS)

    @functools.partial(
        pl.kernel, mesh=mesh,
        out_type=jax.ShapeDtypeStruct((NC, NPAD, LANES), jnp.float32),
        scratch_types=[
            pltpu.VMEM((MAXCH, CHUNK), jnp.int32),
            pltpu.VMEM((CHUNK, LANES), jnp.float32),
            pltpu.VMEM((RPS, LANES), jnp.float32),
            pltpu.VMEM_SHARED((NPAD, LANES), jnp.float32),
        ],
        compiler_params=pltpu.CompilerParams(use_tc_tiling_on_sc=False))
    def k(dst_hbm, out_hbm, dstv, onesb, zbuf, acc):
        c = lax.axis_index("c")
        s = lax.axis_index("s")
        wid = c * NS + s
        start, end = _worker_range(wid, CN)

        def fill_zero(i, carry):
            zbuf[i, :] = jnp.zeros((LANES,), jnp.float32)
            return carry
        lax.fori_loop(0, RPS, fill_zero, None)

        def fill_one(i, carry):
            onesb[i, :] = jnp.ones((LANES,), jnp.float32)
            return carry
        lax.fori_loop(0, CHUNK, fill_one, None)

        pltpu.sync_copy(zbuf, acc.at[pl.ds(s * RPS, RPS)])
        pltpu.sync_copy(dst_hbm.at[pl.ds(start, MAXCH)], dstv)
        plsc.subcore_barrier()

        def body(j, carry):
            pltpu.sync_copy(onesb, acc.at[dstv.at[j]], add=True)
            return carry
        lax.fori_loop(0, end - start, body, None)

        plsc.subcore_barrier()
        pltpu.sync_copy(acc.at[pl.ds(s * RPS, RPS)],
                        out_hbm.at[c, pl.ds(s * RPS, RPS)])

    return k(dst_idx)


def _sc_scatter_rows(src_idx, dst_idx, table):
    CN = dst_idx.shape[0]
    MAXCH = -(-CN // NW)
    mesh = plsc.VectorSubcoreMesh(core_axis_name="c", subcore_axis_name="s",
                                  num_cores=NC, num_subcores=NS)

    @functools.partial(
        pl.kernel, mesh=mesh,
        out_type=jax.ShapeDtypeStruct((NC, NPAD, LANES), jnp.float32),
        scratch_types=[
            pltpu.VMEM((MAXCH, CHUNK), jnp.int32),
            pltpu.VMEM((MAXCH, CHUNK), jnp.int32),
            pltpu.VMEM((CHUNK, LANES), jnp.float32),
            pltpu.VMEM((RPS, LANES), jnp.float32),
            pltpu.VMEM_SHARED((NPAD, LANES), jnp.float32),
            pltpu.VMEM_SHARED((NPAD, LANES), jnp.float32),
            pltpu.SemaphoreType.DMA,
        ],
        compiler_params=pltpu.CompilerParams(use_tc_tiling_on_sc=False))
    def k(src_hbm, dst_hbm, tab_hbm, out_hbm, srcv, dstv, rows, zbuf,
          acc, tabs, gsem):
        c = lax.axis_index("c")
        s = lax.axis_index("s")
        wid = c * NS + s
        start, end = _worker_range(wid, CN)

        def fill_zero(i, carry):
            zbuf[i, :] = jnp.zeros((LANES,), jnp.float32)
            return carry
        lax.fori_loop(0, RPS, fill_zero, None)

        pltpu.sync_copy(zbuf, acc.at[pl.ds(s * RPS, RPS)])
        pltpu.sync_copy(tab_hbm.at[pl.ds(s * RPS, RPS)],
                        tabs.at[pl.ds(s * RPS, RPS)])
        pltpu.sync_copy(src_hbm.at[pl.ds(start, MAXCH)], srcv)
        pltpu.sync_copy(dst_hbm.at[pl.ds(start, MAXCH)], dstv)
        plsc.subcore_barrier()

        def body(j, carry):
            pltpu.async_copy(tabs.at[srcv.at[j]], rows, gsem).wait()
            pltpu.sync_copy(rows, acc.at[dstv.at[j]], add=True)
            return carry
        lax.fori_loop(0, end - start, body, None)

        plsc.subcore_barrier()
        pltpu.sync_copy(acc.at[pl.ds(s * RPS, RPS)],
                        out_hbm.at[c, pl.ds(s * RPS, RPS)])

    return k(src_idx, dst_idx, table)


def _tc_prep(x, W1, dacc):
    def body(x_ref, w_ref, da_ref, db_ref, dinv_ref, dhv_ref):
        deg = da_ref[...] + db_ref[...] + 1.0
        rows = lax.broadcasted_iota(jnp.int32, (NPAD, HID), 0)
        dinv = jnp.where(rows < N, lax.rsqrt(deg), 0.0)
        h = jnp.dot(x_ref[...], w_ref[...], preferred_element_type=jnp.float32)
        hp = jnp.concatenate(
            [h, jnp.zeros((NPAD - N, HID), jnp.float32)], axis=0)
        dinv_ref[...] = dinv
        dhv_ref[...] = dinv * hp

    return pl.pallas_call(
        body,
        out_shape=(jax.ShapeDtypeStruct((NPAD, HID), jnp.float32),
                   jax.ShapeDtypeStruct((NPAD, HID), jnp.float32)),
    )(x, W1, dacc[0], dacc[1])


def _tc_mid(acc, dhv, dinv, b, g, be, W):
    def body(a_ref, b2_ref, dhv_ref, dinv_ref, bias_ref, g_ref, be_ref, w_ref,
             out_ref):
        rows = lax.broadcasted_iota(jnp.int32, (NPAD, HID), 0)
        valid = rows < N
        dinv = dinv_ref[...]
        s = dinv * (a_ref[...] + b2_ref[...] + dhv_ref[...]) + bias_ref[...]
        sv = jnp.where(valid, s, 0.0)
        mean = jnp.sum(sv, axis=0, keepdims=True) * (1.0 / N)
        d = s - mean
        var = jnp.sum(jnp.where(valid, d * d, 0.0), axis=0, keepdims=True) * (1.0 / N)
        bn = d * lax.rsqrt(var + EPS) * g_ref[...] + be_ref[...]
        h = jnp.where(valid, jnp.maximum(bn, 0.0), 0.0)
        out_ref[...] = dinv * jnp.dot(h, w_ref[...],
                                      preferred_element_type=jnp.float32)

    return pl.pallas_call(
        body,
        out_shape=jax.ShapeDtypeStruct((NPAD, HID), jnp.float32),
    )(acc[0], acc[1], dhv, dinv, b, g, be, W)


def _tc_final(acc, dhv, dinv, b, g, be, Wfc, bfc):
    def body(a_ref, b2_ref, dhv_ref, dinv_ref, bias_ref, g_ref, be_ref, w_ref,
             bf_ref, out_ref):
        rows = lax.broadcasted_iota(jnp.int32, (NPAD, HID), 0)
        valid = rows < N
        s = dinv_ref[...] * (a_ref[...] + b2_ref[...] + dhv_ref[...]) + bias_ref[...]
        sv = jnp.where(valid, s, 0.0)
        mean = jnp.sum(sv, axis=0, keepdims=True) * (1.0 / N)
        d = s - mean
        var = jnp.sum(jnp.where(valid, d * d, 0.0), axis=0, keepdims=True) * (1.0 / N)
        bn = d * lax.rsqrt(var + EPS) * g_ref[...] + be_ref[...]
        h = jnp.maximum(bn[:N], 0.0)
        out_ref[...] = jnp.dot(h, w_ref[...],
                               preferred_element_type=jnp.float32) + bf_ref[...]

    return pl.pallas_call(
        body,
        out_shape=jax.ShapeDtypeStruct((N, OUT_DIM), jnp.float32),
    )(acc[0], acc[1], dhv, dinv, b, g, be, Wfc, bfc)


def kernel(x, edge_index, W1, b1, g1, be1, W2, b2, g2, be2, Wfc, bfc):
    E = edge_index.shape[1]
    ei = edge_index.astype(jnp.int32)
    if E % CHUNK:
        padn = CHUNK - E % CHUNK
        ei = jnp.concatenate(
            [ei, jnp.full((2, padn), NPAD - 1, jnp.int32)], axis=1)
    CN = ei.shape[1] // CHUNK
    src = ei[0].reshape(CN, CHUNK)
    dst = ei[1].reshape(CN, CHUNK)

    dacc = _sc_degree(dst)
    dinv, ht1 = _tc_prep(x, W1, dacc)
    acc1 = _sc_scatter_rows(src, dst, ht1)
    ht2 = _tc_mid(acc1, ht1, dinv, b1.reshape(1, -1), g1.reshape(1, -1),
                  be1.reshape(1, -1), W2)
    acc2 = _sc_scatter_rows(src, dst, ht2)
    out = _tc_final(acc2, ht2, dinv, b2.reshape(1, -1), g2.reshape(1, -1),
                    be2.reshape(1, -1), Wfc, bfc.reshape(1, -1))
    return out


# speedup vs baseline: 4.3549x; 1.0536x over previous
import functools

import jax
import jax.numpy as jnp
from jax import lax
from jax.experimental import pallas as pl
from jax.experimental.pallas import tpu as pltpu
from jax.experimental.pallas import tpu_sc as plsc

N = 10000
HID = 16
OUT_DIM = 64
EPS = 1e-5

NC, NS, LANES = 2, 16, 16
NW = NC * NS
CHUNK = 128
RPS = 632
NPAD = NS * RPS


def _worker_range(wid, cn):
    start = wid * cn // NW
    end = (wid + 1) * cn // NW
    return start, end


def _sc_degree(dst_idx):
    CN = dst_idx.shape[0]
    MAXCH = -(-CN // NW)
    mesh = plsc.VectorSubcoreMesh(core_axis_name="c", subcore_axis_name="s",
                                  num_cores=NC, num_subcores=NS)

    @functools.partial(
        pl.kernel, mesh=mesh,
        out_type=jax.ShapeDtypeStruct((NC, NPAD, LANES), jnp.float32),
        scratch_types=[
            pltpu.VMEM((MAXCH, CHUNK), jnp.int32),
            pltpu.VMEM((CHUNK, LANES), jnp.float32),
            pltpu.VMEM((RPS, LANES), jnp.float32),
            pltpu.VMEM_SHARED((NPAD, LANES), jnp.float32),
        ],
        compiler_params=pltpu.CompilerParams(use_tc_tiling_on_sc=False))
    def k(dst_hbm, out_hbm, dstv, onesb, zbuf, acc):
        c = lax.axis_index("c")
        s = lax.axis_index("s")
        wid = c * NS + s
        start, end = _worker_range(wid, CN)

        def fill_zero(i, carry):
            zbuf[i, :] = jnp.zeros((LANES,), jnp.float32)
            return carry
        lax.fori_loop(0, RPS, fill_zero, None)

        def fill_one(i, carry):
            onesb[i, :] = jnp.ones((LANES,), jnp.float32)
            return carry
        lax.fori_loop(0, CHUNK, fill_one, None)

        pltpu.sync_copy(zbuf, acc.at[pl.ds(s * RPS, RPS)])
        pltpu.sync_copy(dst_hbm.at[pl.ds(start, MAXCH)], dstv)
        plsc.subcore_barrier()

        def body(j, carry):
            pltpu.sync_copy(onesb, acc.at[dstv.at[j]], add=True)
            return carry
        lax.fori_loop(0, end - start, body, None)

        plsc.subcore_barrier()
        pltpu.sync_copy(acc.at[pl.ds(s * RPS, RPS)],
                        out_hbm.at[c, pl.ds(s * RPS, RPS)])

    return k(dst_idx)


def _sc_scatter_rows(src_idx, dst_idx, table):
    CN = dst_idx.shape[0]
    MAXCH = -(-CN // NW)
    mesh = plsc.VectorSubcoreMesh(core_axis_name="c", subcore_axis_name="s",
                                  num_cores=NC, num_subcores=NS)

    @functools.partial(
        pl.kernel, mesh=mesh,
        out_type=jax.ShapeDtypeStruct((NC, NPAD, LANES), jnp.float32),
        scratch_types=[
            pltpu.VMEM((MAXCH, CHUNK), jnp.int32),
            pltpu.VMEM((MAXCH, CHUNK), jnp.int32),
            pltpu.VMEM((CHUNK, LANES), jnp.float32),
            pltpu.VMEM((RPS, LANES), jnp.float32),
            pltpu.VMEM_SHARED((NPAD, LANES), jnp.float32),
            pltpu.VMEM_SHARED((NPAD, LANES), jnp.float32),
            pltpu.SemaphoreType.DMA,
        ],
        compiler_params=pltpu.CompilerParams(use_tc_tiling_on_sc=False))
    def k(src_hbm, dst_hbm, tab_hbm, out_hbm, srcv, dstv, rows, zbuf,
          acc, tabs, gsem):
        c = lax.axis_index("c")
        s = lax.axis_index("s")
        wid = c * NS + s
        start, end = _worker_range(wid, CN)

        def fill_zero(i, carry):
            zbuf[i, :] = jnp.zeros((LANES,), jnp.float32)
            return carry
        lax.fori_loop(0, RPS, fill_zero, None)

        pltpu.sync_copy(zbuf, acc.at[pl.ds(s * RPS, RPS)])
        pltpu.sync_copy(tab_hbm.at[pl.ds(s * RPS, RPS)],
                        tabs.at[pl.ds(s * RPS, RPS)])
        pltpu.sync_copy(src_hbm.at[pl.ds(start, MAXCH)], srcv)
        pltpu.sync_copy(dst_hbm.at[pl.ds(start, MAXCH)], dstv)
        plsc.subcore_barrier()

        def body(j, carry):
            pltpu.async_copy(tabs.at[srcv.at[j]], rows, gsem).wait()
            pltpu.sync_copy(rows, acc.at[dstv.at[j]], add=True)
            return carry
        lax.fori_loop(0, end - start, body, None)

        plsc.subcore_barrier()
        pltpu.sync_copy(acc.at[pl.ds(s * RPS, RPS)],
                        out_hbm.at[c, pl.ds(s * RPS, RPS)])

    return k(src_idx, dst_idx, table)


def _sc_scatter_scaled(src_idx, dst_idx, h1, dacc):
    CN = dst_idx.shape[0]
    MAXCH = -(-CN // NW)
    mesh = plsc.VectorSubcoreMesh(core_axis_name="c", subcore_axis_name="s",
                                  num_cores=NC, num_subcores=NS)

    @functools.partial(
        pl.kernel, mesh=mesh,
        out_type=jax.ShapeDtypeStruct((NC, NPAD, LANES), jnp.float32),
        scratch_types=[
            pltpu.VMEM((MAXCH, CHUNK), jnp.int32),
            pltpu.VMEM((MAXCH, CHUNK), jnp.int32),
            pltpu.VMEM((CHUNK, LANES), jnp.float32),
            pltpu.VMEM((RPS, LANES), jnp.float32),
            pltpu.VMEM((RPS, LANES), jnp.float32),
            pltpu.VMEM((RPS, LANES), jnp.float32),
            pltpu.VMEM((RPS, LANES), jnp.float32),
            pltpu.VMEM_SHARED((NPAD, LANES), jnp.float32),
            pltpu.VMEM_SHARED((NPAD, LANES), jnp.float32),
            pltpu.SemaphoreType.DMA,
        ],
        compiler_params=pltpu.CompilerParams(use_tc_tiling_on_sc=False))
    def k(src_hbm, dst_hbm, h_hbm, dacc_hbm, out_hbm, srcv, dstv, rows, zbuf,
          abuf, bbuf, hbuf, acc, tabs, gsem):
        c = lax.axis_index("c")
        s = lax.axis_index("s")
        wid = c * NS + s
        start, end = _worker_range(wid, CN)
        sl = pl.ds(s * RPS, RPS)

        def fill_zero(i, carry):
            zbuf[i, :] = jnp.zeros((LANES,), jnp.float32)
            return carry
        lax.fori_loop(0, RPS, fill_zero, None)

        pltpu.sync_copy(zbuf, acc.at[sl])
        pltpu.sync_copy(h_hbm.at[sl], hbuf)
        pltpu.sync_copy(dacc_hbm.at[0, sl], abuf)
        pltpu.sync_copy(dacc_hbm.at[1, sl], bbuf)
        pltpu.sync_copy(src_hbm.at[pl.ds(start, MAXCH)], srcv)
        pltpu.sync_copy(dst_hbm.at[pl.ds(start, MAXCH)], dstv)

        def scale(r, carry):
            deg = abuf[r, :] + bbuf[r, :] + 1.0
            i = lax.bitcast_convert_type(deg, jnp.int32)
            i = jnp.int32(0x5F3759DF) - lax.shift_right_logical(i, 1)
            y = lax.bitcast_convert_type(i, jnp.float32)
            y = y * (1.5 - 0.5 * deg * y * y)
            y = y * (1.5 - 0.5 * deg * y * y)
            y = y * (1.5 - 0.5 * deg * y * y)
            hbuf[r, :] = y * hbuf[r, :]
            return carry
        lax.fori_loop(0, RPS, scale, None)

        pltpu.sync_copy(hbuf, tabs.at[sl])
        plsc.subcore_barrier()

        def body(j, carry):
            pltpu.async_copy(tabs.at[srcv.at[j]], rows, gsem).wait()
            pltpu.sync_copy(rows, acc.at[dstv.at[j]], add=True)
            return carry
        lax.fori_loop(0, end - start, body, None)

        plsc.subcore_barrier()
        pltpu.sync_copy(acc.at[sl], out_hbm.at[c, sl])

    return k(src_idx, dst_idx, h1, dacc)


def _tc_matmul(x, W1):
    def body(x_ref, w_ref, h_ref):
        h = jnp.dot(x_ref[...], w_ref[...], preferred_element_type=jnp.float32)
        h_ref[...] = jnp.concatenate(
            [h, jnp.zeros((NPAD - N, HID), jnp.float32)], axis=0)

    return pl.pallas_call(
        body,
        out_shape=jax.ShapeDtypeStruct((NPAD, HID), jnp.float32),
    )(x, W1)


def _tc_mid(acc, dacc, h1, b, g, be, W):
    def body(a_ref, b2_ref, da_ref, db_ref, h1_ref, bias_ref, g_ref, be_ref,
             w_ref, out_ref):
        rows = lax.broadcasted_iota(jnp.int32, (NPAD, HID), 0)
        valid = rows < N
        deg = da_ref[...] + db_ref[...] + 1.0
        dinv = jnp.where(valid, lax.rsqrt(deg), 0.0)
        s = dinv * (a_ref[...] + b2_ref[...] + dinv * h1_ref[...]) + bias_ref[...]
        sv = jnp.where(valid, s, 0.0)
        mean = jnp.sum(sv, axis=0, keepdims=True) * (1.0 / N)
        d = s - mean
        var = jnp.sum(jnp.where(valid, d * d, 0.0), axis=0, keepdims=True) * (1.0 / N)
        bn = d * lax.rsqrt(var + EPS) * g_ref[...] + be_ref[...]
        h = jnp.where(valid, jnp.maximum(bn, 0.0), 0.0)
        out_ref[...] = dinv * jnp.dot(h, w_ref[...],
                                      preferred_element_type=jnp.float32)

    return pl.pallas_call(
        body,
        out_shape=jax.ShapeDtypeStruct((NPAD, HID), jnp.float32),
    )(acc[0], acc[1], dacc[0], dacc[1], h1, b, g, be, W)


def _tc_final(acc, dacc, dhv, b, g, be, Wfc, bfc):
    def body(a_ref, b2_ref, da_ref, db_ref, dhv_ref, bias_ref, g_ref, be_ref,
             w_ref, bf_ref, out_ref):
        rows = lax.broadcasted_iota(jnp.int32, (NPAD, HID), 0)
        valid = rows < N
        deg = da_ref[...] + db_ref[...] + 1.0
        dinv = jnp.where(valid, lax.rsqrt(deg), 0.0)
        s = dinv * (a_ref[...] + b2_ref[...] + dhv_ref[...]) + bias_ref[...]
        sv = jnp.where(valid, s, 0.0)
        mean = jnp.sum(sv, axis=0, keepdims=True) * (1.0 / N)
        d = s - mean
        var = jnp.sum(jnp.where(valid, d * d, 0.0), axis=0, keepdims=True) * (1.0 / N)
        bn = d * lax.rsqrt(var + EPS) * g_ref[...] + be_ref[...]
        h = jnp.maximum(bn[:N], 0.0)
        out_ref[...] = jnp.dot(h, w_ref[...],
                               preferred_element_type=jnp.float32) + bf_ref[...]

    return pl.pallas_call(
        body,
        out_shape=jax.ShapeDtypeStruct((N, OUT_DIM), jnp.float32),
    )(acc[0], acc[1], dacc[0], dacc[1], dhv, b, g, be, Wfc, bfc)


def kernel(x, edge_index, W1, b1, g1, be1, W2, b2, g2, be2, Wfc, bfc):
    E = edge_index.shape[1]
    ei = edge_index.astype(jnp.int32)
    if E % CHUNK:
        padn = CHUNK - E % CHUNK
        ei = jnp.concatenate(
            [ei, jnp.full((2, padn), NPAD - 1, jnp.int32)], axis=1)
    CN = ei.shape[1] // CHUNK
    src = ei[0].reshape(CN, CHUNK)
    dst = ei[1].reshape(CN, CHUNK)

    h1 = _tc_matmul(x, W1)
    dacc = _sc_degree(dst)
    acc1 = _sc_scatter_scaled(src, dst, h1, dacc)
    ht2 = _tc_mid(acc1, dacc, h1, b1.reshape(1, -1), g1.reshape(1, -1),
                  be1.reshape(1, -1), W2)
    acc2 = _sc_scatter_rows(src, dst, ht2)
    out = _tc_final(acc2, dacc, ht2, b2.reshape(1, -1), g2.reshape(1, -1),
                    be2.reshape(1, -1), Wfc, bfc.reshape(1, -1))
    return out
